# baseline (speedup 1.0000x reference)
import functools

import jax
import jax.numpy as jnp
from jax import lax
from jax.experimental import pallas as pl
from jax.experimental.pallas import tpu as pltpu
from jax.experimental.pallas import tpu_sc as plsc

B, C, H, W = 16, 256, 32, 32
HW = H * W
N = B * HW
K = 1024
D = C
L = 16

NC = 2
NS = 16
NW = NC * NS
CPT = C // NW


def _tc_body(emb_ref, z_ref, idx_ref, stats_ref, embt_ref, counts_ref, acc_ref):
    b = pl.program_id(0)
    emb = emb_ref[...]
    zb = z_ref[0]

    @pl.when(b == 0)
    def _embt():
        embt_ref[...] = jnp.transpose(emb)
    s_t = jax.lax.dot_general(
        emb, zb, (((1,), (0,)), ((), ())),
        preferred_element_type=jnp.float32,
        precision=lax.Precision.DEFAULT,
    )
    enorm = jnp.sum(emb * emb, axis=1, keepdims=True)
    znorm = jnp.sum(zb * zb, axis=0, keepdims=True)
    dist_t = (znorm - 2.0 * s_t) + enorm
    m = jnp.min(dist_t, axis=0, keepdims=True)
    ks = lax.broadcasted_iota(jnp.int32, (K, HW), 0)
    eq = dist_t == m
    idx = jnp.min(jnp.where(eq, ks, K), axis=0).astype(jnp.int32)
    idx_ref[0, 0, :] = idx

    @pl.when(b == 0)
    def _init():
        counts_ref[...] = jnp.zeros_like(counts_ref)
        acc_ref[0] = 0.0

    ones = jnp.ones((HW, 1), jnp.float32)
    counts_ref[...] += jax.lax.dot_general(
        eq.astype(jnp.float32), ones, (((1,), (0,)), ((), ())),
        preferred_element_type=jnp.float32)
    esum = jnp.sum(emb, axis=0, keepdims=True)
    zsum = jnp.sum(zb, axis=1, keepdims=True)
    cross = jax.lax.dot_general(
        esum, zsum, (((1,), (0,)), ((), ())),
        preferred_element_type=jnp.float32,
        precision=lax.Precision.HIGHEST)
    acc_ref[0] += (K * jnp.sum(znorm) + HW * jnp.sum(enorm)
                   - 2.0 * cross[0, 0])

    @pl.when(b == B - 1)
    def _finalize():
        e_mean = counts_ref[...] * (1.0 / N)
        ent = jnp.sum(e_mean * jnp.log(e_mean + 1e-10))
        stats_ref[0] = jnp.exp(-ent)
        stats_ref[1] = acc_ref[0] * (1.0 / (N * K))


_tc_call = pl.pallas_call(
    _tc_body,
    grid=(B,),
    in_specs=[
        pl.BlockSpec((K, D), lambda b: (0, 0)),
        pl.BlockSpec((1, C, HW), lambda b: (b, 0, 0)),
    ],
    out_specs=[
        pl.BlockSpec((1, 1, HW), lambda b: (b, 0, 0)),
        pl.BlockSpec(memory_space=pltpu.SMEM),
        pl.BlockSpec((D, K), lambda b: (0, 0)),
    ],
    out_shape=[
        jax.ShapeDtypeStruct((B, 1, HW), jnp.int32),
        jax.ShapeDtypeStruct((2,), jnp.float32),
        jax.ShapeDtypeStruct((D, K), jnp.float32),
    ],
    scratch_shapes=[
        pltpu.VMEM((K, 1), jnp.float32),
        pltpu.SMEM((2,), jnp.float32),
    ],
)


def _sc_zq_body(embt_hbm, idx_hbm, out_hbm, embt_v, idx_v, ob0, ob1, so0, so1):
    c = lax.axis_index("c")
    s = lax.axis_index("s")
    wid = s * NC + c
    crow = wid * CPT
    pltpu.sync_copy(embt_hbm.at[pl.ds(crow * K, CPT * K)], embt_v)
    pltpu.sync_copy(idx_hbm, idx_v)

    def fill(b, ob):
        @pl.loop(0, HW // L, unroll=8)
        def _g(g):
            iv = idx_v[pl.ds(b * HW + g * L, L)]
            for ci in range(CPT):
                ob[pl.ds(ci * HW + g * L, L)] = plsc.load_gather(
                    embt_v, [iv + (ci * K)])

    @pl.loop(0, B // 2)
    def _k(k):
        for par, ob, so in ((0, ob0, so0), (1, ob1, so1)):
            b = 2 * k + par

            @pl.when(k > 0)
            def _wait_prev():
                pltpu.make_async_copy(
                    ob, out_hbm.at[0].at[pl.ds(crow * HW, CPT * HW)], so).wait()

            fill(b, ob)
            pltpu.async_copy(
                ob, out_hbm.at[b].at[pl.ds(crow * HW, CPT * HW)], so)

    pltpu.make_async_copy(
        ob0, out_hbm.at[0].at[pl.ds(crow * HW, CPT * HW)], so0).wait()
    pltpu.make_async_copy(
        ob1, out_hbm.at[1].at[pl.ds(crow * HW, CPT * HW)], so1).wait()


@functools.lru_cache(maxsize=1)
def _make_sc_zq():
    return pl.kernel(
        _sc_zq_body,
        out_type=jax.ShapeDtypeStruct((B, C * HW), jnp.float32),
        mesh=plsc.VectorSubcoreMesh(
            core_axis_name="c", subcore_axis_name="s",
            num_cores=NC, num_subcores=NS),
        scratch_types=[
            pltpu.VMEM((CPT * K,), jnp.float32),
            pltpu.VMEM((N,), jnp.int32),
            pltpu.VMEM((CPT * HW,), jnp.float32),
            pltpu.VMEM((CPT * HW,), jnp.float32),
            pltpu.SemaphoreType.DMA,
            pltpu.SemaphoreType.DMA,
        ],
        compiler_params=pltpu.CompilerParams(needs_layout_passes=False),
    )


def kernel(z, embedding):
    zs = z.reshape(B, C, HW)
    idx3, stats, embt = _tc_call(embedding, zs)
    zq = _make_sc_zq()(embt.reshape(D * K), idx3.reshape(N))
    z_q = zq.reshape(B, C, H, W)
    loss = jnp.zeros((), jnp.float32)
    indices = idx3.reshape(B, H, W)
    return (z_q, loss, stats[0], indices, stats[1])


# speedup vs baseline: 1.1767x; 1.1767x over previous
import functools

import jax
import jax.numpy as jnp
from jax import lax
from jax.experimental import pallas as pl
from jax.experimental.pallas import tpu as pltpu
from jax.experimental.pallas import tpu_sc as plsc

B, C, H, W = 16, 256, 32, 32
HW = H * W
N = B * HW
K = 1024
D = C
L = 16

NC = 2
NS = 16
NW = NC * NS
CPT = C // NW


def _tc_body(emb_ref, z_ref, idx_ref, stats_ref, embt_ref, counts_ref, acc_ref):
    b = pl.program_id(0)
    emb = emb_ref[...]
    zb = z_ref[0]

    @pl.when(b == 0)
    def _embt():
        embt_ref[...] = jnp.transpose(emb)
    s_t = jax.lax.dot_general(
        emb, zb, (((1,), (0,)), ((), ())),
        preferred_element_type=jnp.float32,
        precision=lax.Precision.DEFAULT,
    )
    enorm = jnp.sum(emb * emb, axis=1, keepdims=True)
    znorm = jnp.sum(zb * zb, axis=0, keepdims=True)
    dist_t = (znorm - 2.0 * s_t) + enorm
    m = jnp.min(dist_t, axis=0, keepdims=True)
    ks = lax.broadcasted_iota(jnp.int32, (K, HW), 0)
    eq = dist_t == m
    idx = jnp.min(jnp.where(eq, ks, K), axis=0).astype(jnp.int32)
    idx_ref[0, 0, :] = idx

    @pl.when(b == 0)
    def _init():
        counts_ref[...] = jnp.zeros_like(counts_ref)
        acc_ref[0] = 0.0

    ones = jnp.ones((HW, 1), jnp.float32)
    counts_ref[...] += jax.lax.dot_general(
        eq.astype(jnp.float32), ones, (((1,), (0,)), ((), ())),
        preferred_element_type=jnp.float32)
    esum = jnp.sum(emb, axis=0, keepdims=True)
    zsum = jnp.sum(zb, axis=1, keepdims=True)
    cross = jax.lax.dot_general(
        esum, zsum, (((1,), (0,)), ((), ())),
        preferred_element_type=jnp.float32,
        precision=lax.Precision.HIGHEST)
    acc_ref[0] += (K * jnp.sum(znorm) + HW * jnp.sum(enorm)
                   - 2.0 * cross[0, 0])

    @pl.when(b == B - 1)
    def _finalize():
        e_mean = counts_ref[...] * (1.0 / N)
        ent = jnp.sum(e_mean * jnp.log(e_mean + 1e-10))
        stats_ref[0] = jnp.exp(-ent)
        stats_ref[1] = acc_ref[0] * (1.0 / (N * K))


_tc_call = pl.pallas_call(
    _tc_body,
    grid=(B,),
    in_specs=[
        pl.BlockSpec((K, D), lambda b: (0, 0)),
        pl.BlockSpec((1, C, HW), lambda b: (b, 0, 0)),
    ],
    out_specs=[
        pl.BlockSpec((1, 1, HW), lambda b: (b, 0, 0)),
        pl.BlockSpec(memory_space=pltpu.SMEM),
        pl.BlockSpec((D, K), lambda b: (0, 0)),
    ],
    out_shape=[
        jax.ShapeDtypeStruct((B, 1, HW), jnp.int32),
        jax.ShapeDtypeStruct((2,), jnp.float32),
        jax.ShapeDtypeStruct((D, K), jnp.float32),
    ],
    scratch_shapes=[
        pltpu.VMEM((K, 1), jnp.float32),
        pltpu.SMEM((2,), jnp.float32),
    ],
)


def _sc_zq_body(embt_hbm, idx_hbm, out_hbm, embt_v, idx_v, ob0, ob1, so0, so1):
    c = lax.axis_index("c")
    s = lax.axis_index("s")
    wid = s * NC + c
    crow = wid * CPT
    pltpu.sync_copy(embt_hbm.at[pl.ds(crow * K, CPT * K)], embt_v)
    pltpu.sync_copy(idx_hbm, idx_v)

    def fill(b, ob):
        @plsc.parallel_loop(0, HW // L, unroll=8)
        def _g(g):
            iv = idx_v[pl.ds(b * HW + g * L, L)]
            for ci in range(CPT):
                ob[pl.ds(ci * HW + g * L, L)] = plsc.load_gather(
                    embt_v, [iv + (ci * K)])

    @pl.loop(0, B // 2)
    def _k(k):
        for par, ob, so in ((0, ob0, so0), (1, ob1, so1)):
            b = 2 * k + par

            @pl.when(k > 0)
            def _wait_prev():
                pltpu.make_async_copy(
                    ob, out_hbm.at[0].at[pl.ds(crow * HW, CPT * HW)], so).wait()

            fill(b, ob)
            pltpu.async_copy(
                ob, out_hbm.at[b].at[pl.ds(crow * HW, CPT * HW)], so)

    pltpu.make_async_copy(
        ob0, out_hbm.at[0].at[pl.ds(crow * HW, CPT * HW)], so0).wait()
    pltpu.make_async_copy(
        ob1, out_hbm.at[1].at[pl.ds(crow * HW, CPT * HW)], so1).wait()


@functools.lru_cache(maxsize=1)
def _make_sc_zq():
    return pl.kernel(
        _sc_zq_body,
        out_type=jax.ShapeDtypeStruct((B, C * HW), jnp.float32),
        mesh=plsc.VectorSubcoreMesh(
            core_axis_name="c", subcore_axis_name="s",
            num_cores=NC, num_subcores=NS),
        scratch_types=[
            pltpu.VMEM((CPT * K,), jnp.float32),
            pltpu.VMEM((N,), jnp.int32),
            pltpu.VMEM((CPT * HW,), jnp.float32),
            pltpu.VMEM((CPT * HW,), jnp.float32),
            pltpu.SemaphoreType.DMA,
            pltpu.SemaphoreType.DMA,
        ],
        compiler_params=pltpu.CompilerParams(needs_layout_passes=False),
    )


def kernel(z, embedding):
    zs = z.reshape(B, C, HW)
    idx3, stats, embt = _tc_call(embedding, zs)
    zq = _make_sc_zq()(embt.reshape(D * K), idx3.reshape(N))
    z_q = zq.reshape(B, C, H, W)
    loss = jnp.zeros((), jnp.float32)
    indices = idx3.reshape(B, H, W)
    return (z_q, loss, stats[0], indices, stats[1])


# speedup vs baseline: 1.2477x; 1.0604x over previous
import functools

import jax
import jax.numpy as jnp
from jax import lax
from jax.experimental import pallas as pl
from jax.experimental.pallas import tpu as pltpu
from jax.experimental.pallas import tpu_sc as plsc

B, C, H, W = 16, 256, 32, 32
HW = H * W
N = B * HW
K = 1024
D = C
L = 16

NC = 2
NS = 16
NW = NC * NS
CPT = C // NW


def _tc_body(emb_ref, z_ref, idx_ref, stats_ref, embt_ref, counts_ref, acc_ref):
    b = pl.program_id(0)
    emb = emb_ref[...]
    zb = z_ref[0]

    @pl.when(b == 0)
    def _embt():
        for j in range(K // 128):
            embt_ref[j] = jnp.transpose(emb[j * 128:(j + 1) * 128, :])
    s_t = jax.lax.dot_general(
        emb, zb, (((1,), (0,)), ((), ())),
        preferred_element_type=jnp.float32,
        precision=lax.Precision.DEFAULT,
    )
    enorm = jnp.sum(emb * emb, axis=1, keepdims=True)
    znorm = jnp.sum(zb * zb, axis=0, keepdims=True)
    dist_t = (znorm - 2.0 * s_t) + enorm
    m = jnp.min(dist_t, axis=0, keepdims=True)
    ks = lax.broadcasted_iota(jnp.int32, (K, HW), 0)
    eq = dist_t == m
    idx = jnp.min(jnp.where(eq, ks, K), axis=0).astype(jnp.int32)
    idx_ref[...] = idx.reshape(8, 128)

    @pl.when(b == 0)
    def _init():
        counts_ref[...] = jnp.zeros_like(counts_ref)
        acc_ref[0] = 0.0

    ones = jnp.ones((HW, 1), jnp.float32)
    counts_ref[...] += jax.lax.dot_general(
        eq.astype(jnp.float32), ones, (((1,), (0,)), ((), ())),
        preferred_element_type=jnp.float32)
    esum = jnp.sum(emb, axis=0, keepdims=True)
    zsum = jnp.sum(zb, axis=1, keepdims=True)
    cross = jax.lax.dot_general(
        esum, zsum, (((1,), (0,)), ((), ())),
        preferred_element_type=jnp.float32,
        precision=lax.Precision.HIGHEST)
    acc_ref[0] += (K * jnp.sum(znorm) + HW * jnp.sum(enorm)
                   - 2.0 * cross[0, 0])

    @pl.when(b == B - 1)
    def _finalize():
        e_mean = counts_ref[...] * (1.0 / N)
        ent = jnp.sum(e_mean * jnp.log(e_mean + 1e-10))
        stats_ref[0] = jnp.exp(-ent)
        stats_ref[1] = acc_ref[0] * (1.0 / (N * K))


_tc_call = pl.pallas_call(
    _tc_body,
    grid=(B,),
    in_specs=[
        pl.BlockSpec((K, D), lambda b: (0, 0)),
        pl.BlockSpec((1, C, HW), lambda b: (b, 0, 0)),
    ],
    out_specs=[
        pl.BlockSpec((8, 128), lambda b: (b, 0)),
        pl.BlockSpec(memory_space=pltpu.SMEM),
        pl.BlockSpec((K // 128, D, 128), lambda b: (0, 0, 0)),
    ],
    out_shape=[
        jax.ShapeDtypeStruct((N // 128, 128), jnp.int32),
        jax.ShapeDtypeStruct((2,), jnp.float32),
        jax.ShapeDtypeStruct((K // 128, D, 128), jnp.float32),
    ],
    scratch_shapes=[
        pltpu.VMEM((K, 1), jnp.float32),
        pltpu.SMEM((2,), jnp.float32),
    ],
)


def _sc_zq_body(embt_hbm, idx_hbm, out_hbm, embt_v, idx_v, ob0, ob1, so0, so1):
    c = lax.axis_index("c")
    s = lax.axis_index("s")
    wid = s * NC + c
    crow = wid * CPT
    for j in range(K // 128):
        pltpu.sync_copy(
            embt_hbm.at[pl.ds(j * D * 128 + crow * 128, CPT * 128)],
            embt_v.at[pl.ds(j * CPT * 128, CPT * 128)])
    pltpu.sync_copy(idx_hbm, idx_v)

    def fill(b, ob):
        @plsc.parallel_loop(0, HW // L, unroll=8)
        def _g(g):
            iv = idx_v[pl.ds(b * HW + g * L, L)]
            base = ((iv >> 7) << 10) + (iv & 127)
            h = g >> 1
            w0 = (g & 1) * L
            for ci in range(CPT):
                ob[ci, h, pl.ds(w0, L)] = plsc.load_gather(
                    embt_v, [base + (ci * 128)])

    @pl.loop(0, B // 2)
    def _k(k):
        for par, ob, so in ((0, ob0, so0), (1, ob1, so1)):
            b = 2 * k + par

            @pl.when(k > 0)
            def _wait_prev():
                pltpu.make_async_copy(
                    ob, out_hbm.at[0].at[pl.ds(crow, CPT)], so).wait()

            fill(b, ob)
            pltpu.async_copy(ob, out_hbm.at[b].at[pl.ds(crow, CPT)], so)

    pltpu.make_async_copy(ob0, out_hbm.at[0].at[pl.ds(crow, CPT)], so0).wait()
    pltpu.make_async_copy(ob1, out_hbm.at[1].at[pl.ds(crow, CPT)], so1).wait()


@functools.lru_cache(maxsize=1)
def _make_sc_zq():
    return pl.kernel(
        _sc_zq_body,
        out_type=jax.ShapeDtypeStruct((B, C, H, W), jnp.float32),
        mesh=plsc.VectorSubcoreMesh(
            core_axis_name="c", subcore_axis_name="s",
            num_cores=NC, num_subcores=NS),
        scratch_types=[
            pltpu.VMEM((CPT * K,), jnp.float32),
            pltpu.VMEM((N,), jnp.int32),
            pltpu.VMEM((CPT, H, W), jnp.float32),
            pltpu.VMEM((CPT, H, W), jnp.float32),
            pltpu.SemaphoreType.DMA,
            pltpu.SemaphoreType.DMA,
        ],
        compiler_params=pltpu.CompilerParams(needs_layout_passes=False),
    )


def kernel(z, embedding):
    zs = z.reshape(B, C, HW)
    idx2, stats, embt = _tc_call(embedding, zs)
    z_q = _make_sc_zq()(embt.reshape(D * K), idx2.reshape(N))
    loss = jnp.zeros((), jnp.float32)
    indices = idx2.reshape(B, H, W)
    return (z_q, loss, stats[0], indices, stats[1])


# speedup vs baseline: 1.6953x; 1.3587x over previous
import functools

import jax
import jax.numpy as jnp
from jax import lax
from jax.experimental import pallas as pl
from jax.experimental.pallas import tpu as pltpu
from jax.experimental.pallas import tpu_sc as plsc

B, C, H, W = 16, 256, 32, 32
HW = H * W
N = B * HW
K = 1024
D = C

NC = 2
NS = 16
NW = NC * NS
ROWS_PER_W = N // NW
CHUNK = 128
NCHUNK = ROWS_PER_W // CHUNK
NBUF = 3


def _tc_body(emb_ref, z_ref, idx_ref, stats_ref, counts_ref, acc_ref):
    b = pl.program_id(0)
    emb = emb_ref[...]
    zb = z_ref[0]
    s_t = jax.lax.dot_general(
        emb, zb, (((1,), (0,)), ((), ())),
        preferred_element_type=jnp.float32,
        precision=lax.Precision.DEFAULT,
    )
    enorm = jnp.sum(emb * emb, axis=1, keepdims=True)
    znorm = jnp.sum(zb * zb, axis=0, keepdims=True)
    dist_t = (znorm - 2.0 * s_t) + enorm
    m = jnp.min(dist_t, axis=0, keepdims=True)
    ks = lax.broadcasted_iota(jnp.int32, (K, HW), 0)
    eq = dist_t == m
    idx = jnp.min(jnp.where(eq, ks, K), axis=0).astype(jnp.int32)
    idx_ref[0, 0, :] = idx

    @pl.when(b == 0)
    def _init():
        counts_ref[...] = jnp.zeros_like(counts_ref)
        acc_ref[0] = 0.0

    ones = jnp.ones((HW, 1), jnp.float32)
    counts_ref[...] += jax.lax.dot_general(
        eq.astype(jnp.float32), ones, (((1,), (0,)), ((), ())),
        preferred_element_type=jnp.float32)
    esum = jnp.sum(emb, axis=0, keepdims=True)
    zsum = jnp.sum(zb, axis=1, keepdims=True)
    cross = jax.lax.dot_general(
        esum, zsum, (((1,), (0,)), ((), ())),
        preferred_element_type=jnp.float32,
        precision=lax.Precision.HIGHEST)
    acc_ref[0] += (K * jnp.sum(znorm) + HW * jnp.sum(enorm)
                   - 2.0 * cross[0, 0])

    @pl.when(b == B - 1)
    def _finalize():
        e_mean = counts_ref[...] * (1.0 / N)
        ent = jnp.sum(e_mean * jnp.log(e_mean + 1e-10))
        stats_ref[0] = jnp.exp(-ent)
        stats_ref[1] = acc_ref[0] * (1.0 / (N * K))


_tc_call = pl.pallas_call(
    _tc_body,
    grid=(B,),
    in_specs=[
        pl.BlockSpec((K, D), lambda b: (0, 0)),
        pl.BlockSpec((1, C, HW), lambda b: (b, 0, 0)),
    ],
    out_specs=[
        pl.BlockSpec((1, 1, HW), lambda b: (b, 0, 0)),
        pl.BlockSpec(memory_space=pltpu.SMEM),
    ],
    out_shape=[
        jax.ShapeDtypeStruct((B, 1, HW), jnp.int32),
        jax.ShapeDtypeStruct((2,), jnp.float32),
    ],
    scratch_shapes=[
        pltpu.VMEM((K, 1), jnp.float32),
        pltpu.SMEM((2,), jnp.float32),
    ],
)


def _sc_gather_body(emb_hbm, idx_hbm, out_hbm, idx_v,
                    buf0, buf1, buf2, sg0, sg1, sg2, so0, so1, so2):
    c = lax.axis_index("c")
    s = lax.axis_index("s")
    wid = s * NC + c
    base = wid * ROWS_PER_W
    pltpu.sync_copy(idx_hbm.at[pl.ds(base, ROWS_PER_W)], idx_v)
    bufs = (buf0, buf1, buf2)
    sgs = (sg0, sg1, sg2)
    sos = (so0, so1, so2)
    gathers = [None] * NCHUNK
    outs = [None] * NCHUNK
    for j in range(min(NBUF, NCHUNK)):
        gathers[j] = pltpu.async_copy(
            emb_hbm.at[idx_v.at[pl.ds(j * CHUNK, CHUNK)]],
            bufs[j % NBUF], sgs[j % NBUF])
    for j in range(NCHUNK):
        gathers[j].wait()
        outs[j] = pltpu.async_copy(
            bufs[j % NBUF], out_hbm.at[pl.ds(base + j * CHUNK, CHUNK)],
            sos[j % NBUF])
        nxt = j + NBUF
        if nxt < NCHUNK:
            outs[j].wait()
            gathers[nxt] = pltpu.async_copy(
                emb_hbm.at[idx_v.at[pl.ds(nxt * CHUNK, CHUNK)]],
                bufs[nxt % NBUF], sgs[nxt % NBUF])
    for j in range(max(0, NCHUNK - NBUF), NCHUNK):
        outs[j].wait()


@functools.lru_cache(maxsize=1)
def _make_sc_gather():
    return pl.kernel(
        _sc_gather_body,
        out_type=jax.ShapeDtypeStruct((N, D), jnp.float32),
        mesh=plsc.VectorSubcoreMesh(
            core_axis_name="c", subcore_axis_name="s",
            num_cores=NC, num_subcores=NS),
        scratch_types=[
            pltpu.VMEM((ROWS_PER_W,), jnp.int32),
            pltpu.VMEM((CHUNK, D), jnp.float32),
            pltpu.VMEM((CHUNK, D), jnp.float32),
            pltpu.VMEM((CHUNK, D), jnp.float32),
            pltpu.SemaphoreType.DMA,
            pltpu.SemaphoreType.DMA,
            pltpu.SemaphoreType.DMA,
            pltpu.SemaphoreType.DMA,
            pltpu.SemaphoreType.DMA,
            pltpu.SemaphoreType.DMA,
        ],
    )


def kernel(z, embedding):
    zs = z.reshape(B, C, HW)
    idx3, stats = _tc_call(embedding, zs)
    zq_flat = _make_sc_gather()(embedding, idx3.reshape(N))
    z_q = zq_flat.reshape(B, HW, C).transpose(0, 2, 1).reshape(B, C, H, W)
    loss = jnp.zeros((), jnp.float32)
    indices = idx3.reshape(B, H, W)
    return (z_q, loss, stats[0], indices, stats[1])


# speedup vs baseline: 1.7229x; 1.0163x over previous
import functools

import jax
import jax.numpy as jnp
from jax import lax
from jax.experimental import pallas as pl
from jax.experimental.pallas import tpu as pltpu
from jax.experimental.pallas import tpu_sc as plsc

B, C, H, W = 16, 256, 32, 32
HW = H * W
N = B * HW
K = 1024
D = C

NC = 2
NS = 16
NW = NC * NS
ROWS_PER_W = N // NW
CHUNK = 128
NCHUNK = ROWS_PER_W // CHUNK
NBUF = 3


def _tc_body(emb_ref, z_ref, idx_ref, stats_ref, counts_ref, acc_ref):
    b = pl.program_id(0)
    emb = emb_ref[...]
    zb = z_ref[0]
    s_t = jax.lax.dot_general(
        emb, zb, (((1,), (0,)), ((), ())),
        preferred_element_type=jnp.float32,
        precision=lax.Precision.DEFAULT,
    )
    enorm = jnp.sum(emb * emb, axis=1, keepdims=True)
    znorm = jnp.sum(zb * zb, axis=0, keepdims=True)
    dist_t = (znorm - 2.0 * s_t) + enorm
    m = jnp.min(dist_t, axis=0, keepdims=True)
    ks = lax.broadcasted_iota(jnp.int32, (K, HW), 0)
    eq = dist_t == m
    idx = jnp.argmin(dist_t, axis=0).astype(jnp.int32)
    idx_ref[0, 0, :] = idx

    @pl.when(b == 0)
    def _init():
        counts_ref[...] = jnp.zeros_like(counts_ref)
        acc_ref[0] = 0.0

    ones = jnp.ones((HW, 1), jnp.float32)
    counts_ref[...] += jax.lax.dot_general(
        eq.astype(jnp.float32), ones, (((1,), (0,)), ((), ())),
        preferred_element_type=jnp.float32)
    esum = jnp.sum(emb, axis=0, keepdims=True)
    zsum = jnp.sum(zb, axis=1, keepdims=True)
    cross = jax.lax.dot_general(
        esum, zsum, (((1,), (0,)), ((), ())),
        preferred_element_type=jnp.float32,
        precision=lax.Precision.HIGHEST)
    acc_ref[0] += (K * jnp.sum(znorm) + HW * jnp.sum(enorm)
                   - 2.0 * cross[0, 0])

    @pl.when(b == B - 1)
    def _finalize():
        e_mean = counts_ref[...] * (1.0 / N)
        ent = jnp.sum(e_mean * jnp.log(e_mean + 1e-10))
        stats_ref[0] = jnp.exp(-ent)
        stats_ref[1] = acc_ref[0] * (1.0 / (N * K))


_tc_call = pl.pallas_call(
    _tc_body,
    grid=(B,),
    in_specs=[
        pl.BlockSpec((K, D), lambda b: (0, 0)),
        pl.BlockSpec((1, C, HW), lambda b: (b, 0, 0)),
    ],
    out_specs=[
        pl.BlockSpec((1, 1, HW), lambda b: (b, 0, 0)),
        pl.BlockSpec(memory_space=pltpu.SMEM),
    ],
    out_shape=[
        jax.ShapeDtypeStruct((B, 1, HW), jnp.int32),
        jax.ShapeDtypeStruct((2,), jnp.float32),
    ],
    scratch_shapes=[
        pltpu.VMEM((K, 1), jnp.float32),
        pltpu.SMEM((2,), jnp.float32),
    ],
)


def _sc_gather_body(emb_hbm, idx_hbm, out_hbm, idx_v,
                    buf0, buf1, buf2, sg0, sg1, sg2, so0, so1, so2):
    c = lax.axis_index("c")
    s = lax.axis_index("s")
    wid = s * NC + c
    base = wid * ROWS_PER_W
    pltpu.sync_copy(idx_hbm.at[pl.ds(base, ROWS_PER_W)], idx_v)
    bufs = (buf0, buf1, buf2)
    sgs = (sg0, sg1, sg2)
    sos = (so0, so1, so2)
    gathers = [None] * NCHUNK
    outs = [None] * NCHUNK
    for j in range(min(NBUF, NCHUNK)):
        gathers[j] = pltpu.async_copy(
            emb_hbm.at[idx_v.at[pl.ds(j * CHUNK, CHUNK)]],
            bufs[j % NBUF], sgs[j % NBUF])
    for j in range(NCHUNK):
        gathers[j].wait()
        outs[j] = pltpu.async_copy(
            bufs[j % NBUF], out_hbm.at[pl.ds(base + j * CHUNK, CHUNK)],
            sos[j % NBUF])
        nxt = j + NBUF
        if nxt < NCHUNK:
            outs[j].wait()
            gathers[nxt] = pltpu.async_copy(
                emb_hbm.at[idx_v.at[pl.ds(nxt * CHUNK, CHUNK)]],
                bufs[nxt % NBUF], sgs[nxt % NBUF])
    for j in range(max(0, NCHUNK - NBUF), NCHUNK):
        outs[j].wait()


@functools.lru_cache(maxsize=1)
def _make_sc_gather():
    return pl.kernel(
        _sc_gather_body,
        out_type=jax.ShapeDtypeStruct((N, D), jnp.float32),
        mesh=plsc.VectorSubcoreMesh(
            core_axis_name="c", subcore_axis_name="s",
            num_cores=NC, num_subcores=NS),
        scratch_types=[
            pltpu.VMEM((ROWS_PER_W,), jnp.int32),
            pltpu.VMEM((CHUNK, D), jnp.float32),
            pltpu.VMEM((CHUNK, D), jnp.float32),
            pltpu.VMEM((CHUNK, D), jnp.float32),
            pltpu.SemaphoreType.DMA,
            pltpu.SemaphoreType.DMA,
            pltpu.SemaphoreType.DMA,
            pltpu.SemaphoreType.DMA,
            pltpu.SemaphoreType.DMA,
            pltpu.SemaphoreType.DMA,
        ],
    )


def kernel(z, embedding):
    zs = z.reshape(B, C, HW)
    idx3, stats = _tc_call(embedding, zs)
    zq_flat = _make_sc_gather()(embedding, idx3.reshape(N))
    z_q = zq_flat.reshape(B, HW, C).transpose(0, 2, 1).reshape(B, C, H, W)
    loss = jnp.zeros((), jnp.float32)
    indices = idx3.reshape(B, H, W)
    return (z_q, loss, stats[0], indices, stats[1])
